# baseline (speedup 1.0000x reference)
import jax
import jax.numpy as jnp
from jax import lax
from jax.experimental import pallas as pl
from jax.experimental.pallas import tpu as pltpu
from jax.experimental.pallas import tpu_sc as plsc

_B, _L, _T = 16, 512, 64
_W = 72
_D, _K = _T, _T + 1
_HALF = _L // 2


def _partition_body(emis_ref, len_ref, trans_ref, transT_ref, start_ref,
                    end_ref, out_ref, ee_ref):
    emis = emis_ref[...]
    lens3 = len_ref[...].reshape(1, _B, 1)

    tpos3 = lax.broadcasted_iota(jnp.int32, (_L, _B, _T), 0)
    valid = tpos3 < lens3

    live = jnp.where(valid, jnp.exp(emis), 0.0)
    iota_r = lax.broadcasted_iota(jnp.int32, (_L, _B, _W - _T), 2)
    dump = (tpos3[:, :, :1] == lens3).astype(jnp.float32)
    right = jnp.where(iota_r == 0, dump,
                      jnp.where(iota_r == 1, 1.0, 0.0))
    ee_ref[...] = jnp.concatenate([live, right],
                                  axis=2).astype(jnp.bfloat16)

    e_end = jnp.exp(end_ref[...])
    e_end_col = jnp.transpose(e_end, (1, 0))
    ic = lax.broadcasted_iota(jnp.int32, (_T, _W - _T), 1)
    top = jnp.concatenate(
        [jnp.exp(trans_ref[...]),
         jnp.where(ic == 0, e_end_col, 0.0)], axis=1)
    ir2 = lax.broadcasted_iota(jnp.int32, (_W - _T, _W), 0)
    ic2 = lax.broadcasted_iota(jnp.int32, (_W - _T, _W), 1)
    bottom = ((ir2 <= 1) & (ic2 == _K)).astype(jnp.float32)
    E = jnp.concatenate([top, bottom], axis=0).astype(jnp.bfloat16)

    topT = jnp.concatenate(
        [jnp.exp(transT_ref[...]), jnp.zeros((_T, _W - _T), jnp.float32)],
        axis=1)
    e_end_pad = jnp.concatenate(
        [e_end, jnp.zeros((1, _W - _T), jnp.float32)], axis=1)
    botT = jnp.where(ir2 == 0, jnp.broadcast_to(e_end_pad, (_W - _T, _W)),
                     jnp.where((ir2 == 1) & ((ic2 == _D) | (ic2 == _K)),
                               1.0, 0.0))
    ET = jnp.concatenate([topT, botT], axis=0).astype(jnp.bfloat16)

    iota_w = lax.broadcasted_iota(jnp.int32, (_B, _W), 1)
    u0 = jnp.concatenate(
        [jnp.exp(start_ref[...]) * jnp.exp(emis[0]),
         jnp.zeros((_B, _W - _T), jnp.float32)],
        axis=1).astype(jnp.bfloat16)
    w0 = ((iota_w == _D) | (iota_w == _K)).astype(jnp.bfloat16)

    def step_f(t, u):
        s = lax.dot_general(u, E, (((1,), (0,)), ((), ())),
                            preferred_element_type=jnp.float32)
        return s.astype(jnp.bfloat16) * ee_ref[t]

    def step_b(t, w):
        h = w * ee_ref[t]
        s = lax.dot_general(h, ET, (((1,), (0,)), ((), ())),
                            preferred_element_type=jnp.float32)
        return s.astype(jnp.bfloat16)

    def rescale(a, c):
        m = jnp.max(a.astype(jnp.float32), axis=1, keepdims=True)
        return (a.astype(jnp.float32) / m).astype(jnp.bfloat16), c + jnp.log(m)

    u, w = u0, w0
    for i in range(1, 8):
        u = step_f(i, u)
    for i in range(8):
        w = step_b(511 - i, w)
    zero_c = jnp.zeros((_B, 1), jnp.float32)
    u, cf = rescale(u, zero_c)
    w, cb = rescale(w, zero_c)

    def block(i, carry):
        u, w, cf, cb = carry
        for q in range(8):
            u = step_f(8 + 8 * i + q, u)
            w = step_b(503 - 8 * i - q, w)
        u, cf = rescale(u, cf)
        w, cb = rescale(w, cb)
        return (u, w, cf, cb)

    u, w, cf, cb = lax.fori_loop(0, 31, block, (u, w, cf, cb))

    z = jnp.sum(u.astype(jnp.float32) * w.astype(jnp.float32),
                axis=1, keepdims=True)
    out_ref[...] = cf + cb + jnp.log(z)


def _make_score_kernel():
    mesh = plsc.VectorSubcoreMesh(core_axis_name="c", subcore_axis_name="s")

    def body(emis_hbm, tags_hbm, tagsn_hbm, lenb_hbm, len_hbm, trans_hbm,
             start_hbm, end_hbm, out_hbm, tags_v, tagsn_v, len_v, eidx_v,
             tidx_v, evals_v, tvals_v, acc_v, i16_v, t16_v, f16_v, l16_v):
        c = lax.axis_index("c")
        s = lax.axis_index("s")
        wid = s * 2 + c
        b = wid // 2
        half = wid % 2
        lo = half * _HALF

        pltpu.sync_copy(tags_hbm.at[pl.ds(b * _L + lo, _HALF)], tags_v)
        pltpu.sync_copy(tagsn_hbm.at[pl.ds(b * _L + lo, _HALF)], tagsn_v)
        pltpu.sync_copy(lenb_hbm.at[pl.ds(b * 16, 16)], len_v)
        len_vec = len_v[...]

        base = (b * _L + lo) * _T
        for i in range(_HALF // 16):
            t16 = tags_v[pl.ds(i * 16, 16)]
            tn16 = tagsn_v[pl.ds(i * 16, 16)]
            lane = lax.iota(jnp.int32, 16) + i * 16
            eidx_v[pl.ds(i * 16, 16)] = base + lane * _T + t16
            tidx_v[pl.ds(i * 16, 16)] = t16 * _T + tn16

        pltpu.sync_copy(emis_hbm.at[eidx_v], evals_v)
        pltpu.sync_copy(trans_hbm.at[tidx_v], tvals_v)

        acc = jnp.zeros((16,), jnp.float32)
        for i in range(_HALF // 16):
            t_abs = lax.iota(jnp.int32, 16) + i * 16 + lo
            ev = evals_v[pl.ds(i * 16, 16)]
            tv = tvals_v[pl.ds(i * 16, 16)]
            acc = acc + jnp.where(t_abs < len_vec, ev, 0.0)
            m_tr = ((t_abs + 1) < len_vec) & (t_abs < (_L - 1))
            acc = acc + jnp.where(m_tr, tv, 0.0)
        acc_v[...] = acc
        pltpu.sync_copy(acc_v, out_hbm.at[wid])

        @pl.when(wid == 0)
        def _():
            iota16 = lax.iota(jnp.int32, 16)
            i16_v[...] = iota16 * _L
            pltpu.sync_copy(tags_hbm.at[i16_v], t16_v)
            pltpu.sync_copy(start_hbm.at[t16_v], f16_v)
            pltpu.sync_copy(f16_v, out_hbm.at[32])
            pltpu.sync_copy(len_hbm, l16_v)
            i16_v[...] = iota16 * _L + l16_v[...] - 1
            pltpu.sync_copy(tags_hbm.at[i16_v], t16_v)
            pltpu.sync_copy(end_hbm.at[t16_v], f16_v)
            pltpu.sync_copy(f16_v, out_hbm.at[33])

    return pl.kernel(
        body,
        out_type=jax.ShapeDtypeStruct((34, 16), jnp.float32),
        mesh=mesh,
        scratch_types=[
            pltpu.VMEM((_HALF,), jnp.int32),
            pltpu.VMEM((_HALF,), jnp.int32),
            pltpu.VMEM((16,), jnp.int32),
            pltpu.VMEM((_HALF,), jnp.int32),
            pltpu.VMEM((_HALF,), jnp.int32),
            pltpu.VMEM((_HALF,), jnp.float32),
            pltpu.VMEM((_HALF,), jnp.float32),
            pltpu.VMEM((16,), jnp.float32),
            pltpu.VMEM((16,), jnp.int32),
            pltpu.VMEM((16,), jnp.int32),
            pltpu.VMEM((16,), jnp.float32),
            pltpu.VMEM((16,), jnp.int32),
        ],
    )


def kernel(emissions, tags, lengths, transitions, start_transitions,
           end_transitions):
    lens = jnp.maximum(lengths, 1).astype(jnp.int32)

    tags_flat = tags.reshape(-1)
    tagsn_flat = jnp.concatenate(
        [tags[:, 1:], jnp.zeros((_B, 1), jnp.int32)], axis=1).reshape(-1)
    partials = _make_score_kernel()(
        emissions.reshape(-1), tags_flat, tagsn_flat, jnp.repeat(lens, 16),
        lens, transitions.reshape(-1), start_transitions, end_transitions)
    log_s = (jnp.sum(partials[:32].reshape(_B, 2 * 16), axis=1)
             + partials[32] + partials[33])

    emis_t = jnp.transpose(emissions, (1, 0, 2))
    log_z = pl.pallas_call(
        _partition_body,
        out_shape=jax.ShapeDtypeStruct((_B, 1), jnp.float32),
        scratch_shapes=[pltpu.VMEM((_L, _B, _W), jnp.bfloat16)],
    )(emis_t, lens[:, None], transitions,
      jnp.transpose(transitions, (1, 0)),
      start_transitions[None, :], end_transitions[None, :])

    return log_s - log_z[:, 0]


# speedup vs baseline: 1.0040x; 1.0040x over previous
import jax
import jax.numpy as jnp
from jax import lax
from jax.experimental import pallas as pl
from jax.experimental.pallas import tpu as pltpu
from jax.experimental.pallas import tpu_sc as plsc

_B, _L, _T = 16, 512, 64
_W = 72
_D, _K = _T, _T + 1
_HALF = _L // 2


def _partition_body(emis_ref, len_ref, trans_ref, transT_ref, start_ref,
                    end_ref, out_ref, ee_ref):
    emis = emis_ref[...]
    lens3 = len_ref[...].reshape(1, _B, 1)

    tpos3 = lax.broadcasted_iota(jnp.int32, (_L, _B, _T), 0)
    valid = tpos3 < lens3

    live = jnp.where(valid, jnp.exp(emis), 0.0)
    iota_r = lax.broadcasted_iota(jnp.int32, (_L, _B, _W - _T), 2)
    dump = (tpos3[:, :, :1] == lens3).astype(jnp.float32)
    right = jnp.where(iota_r == 0, dump,
                      jnp.where(iota_r == 1, 1.0, 0.0))
    ee_ref[...] = jnp.concatenate([live, right],
                                  axis=2).astype(jnp.bfloat16)

    e_end = jnp.exp(end_ref[...])
    e_end_col = jnp.transpose(e_end, (1, 0))
    ic = lax.broadcasted_iota(jnp.int32, (_T, _W - _T), 1)
    top = jnp.concatenate(
        [jnp.exp(trans_ref[...]),
         jnp.where(ic == 0, e_end_col, 0.0)], axis=1)
    ir2 = lax.broadcasted_iota(jnp.int32, (_W - _T, _W), 0)
    ic2 = lax.broadcasted_iota(jnp.int32, (_W - _T, _W), 1)
    bottom = ((ir2 <= 1) & (ic2 == _K)).astype(jnp.float32)
    E = jnp.concatenate([top, bottom], axis=0).astype(jnp.bfloat16)

    topT = jnp.concatenate(
        [jnp.exp(transT_ref[...]), jnp.zeros((_T, _W - _T), jnp.float32)],
        axis=1)
    e_end_pad = jnp.concatenate(
        [e_end, jnp.zeros((1, _W - _T), jnp.float32)], axis=1)
    botT = jnp.where(ir2 == 0, jnp.broadcast_to(e_end_pad, (_W - _T, _W)),
                     jnp.where((ir2 == 1) & ((ic2 == _D) | (ic2 == _K)),
                               1.0, 0.0))
    ET = jnp.concatenate([topT, botT], axis=0).astype(jnp.bfloat16)

    iota_w = lax.broadcasted_iota(jnp.int32, (_B, _W), 1)
    u0 = jnp.concatenate(
        [jnp.exp(start_ref[...]) * jnp.exp(emis[0]),
         jnp.zeros((_B, _W - _T), jnp.float32)],
        axis=1).astype(jnp.bfloat16)
    w0 = ((iota_w == _D) | (iota_w == _K)).astype(jnp.bfloat16)

    def step_f(t, u):
        s = lax.dot_general(u, E, (((1,), (0,)), ((), ())),
                            preferred_element_type=jnp.float32)
        return s.astype(jnp.bfloat16) * ee_ref[t]

    def step_b(t, w):
        h = w * ee_ref[t]
        s = lax.dot_general(h, ET, (((1,), (0,)), ((), ())),
                            preferred_element_type=jnp.float32)
        return s.astype(jnp.bfloat16)

    def rescale(a, c):
        m = jnp.max(a.astype(jnp.float32), axis=1, keepdims=True)
        return (a.astype(jnp.float32) / m).astype(jnp.bfloat16), c + jnp.log(m)

    u, w = u0, w0
    for i in range(1, 8):
        u = step_f(i, u)
    for i in range(8):
        w = step_b(511 - i, w)
    zero_c = jnp.zeros((_B, 1), jnp.float32)
    u, cf = rescale(u, zero_c)
    w, cb = rescale(w, zero_c)

    def block(i, carry):
        u, w, cf, cb = carry
        for q in range(8):
            u = step_f(8 + 8 * i + q, u)
            w = step_b(503 - 8 * i - q, w)
        u, cf = rescale(u, cf)
        w, cb = rescale(w, cb)
        return (u, w, cf, cb)

    u, w, cf, cb = lax.fori_loop(0, 31, block, (u, w, cf, cb))

    z = jnp.sum(u.astype(jnp.float32) * w.astype(jnp.float32),
                axis=1, keepdims=True)
    out_ref[...] = cf + cb + jnp.log(z)


def _make_score_kernel():
    mesh = plsc.VectorSubcoreMesh(core_axis_name="c", subcore_axis_name="s")

    def body(emis_hbm, tags_hbm, tagsn_hbm, lenb_hbm, len_hbm, trans_hbm,
             start_hbm, end_hbm, out_hbm, tags_v, tagsn_v, len_v, eidx_v,
             tidx_v, evals_v, tvals_v, acc_v, i16_v, t16_v, f16_v, l16_v,
             sem):
        c = lax.axis_index("c")
        s = lax.axis_index("s")
        wid = s * 2 + c
        b = wid // 2
        half = wid % 2
        lo = half * _HALF

        cp1 = pltpu.async_copy(tags_hbm.at[pl.ds(b * _L + lo, _HALF)],
                               tags_v, sem)
        cp2 = pltpu.async_copy(tagsn_hbm.at[pl.ds(b * _L + lo, _HALF)],
                               tagsn_v, sem)
        cp3 = pltpu.async_copy(lenb_hbm.at[pl.ds(b * 16, 16)], len_v, sem)
        cp1.wait()
        cp2.wait()
        cp3.wait()
        len_vec = len_v[...]

        base = (b * _L + lo) * _T
        for i in range(_HALF // 16):
            t16 = tags_v[pl.ds(i * 16, 16)]
            tn16 = tagsn_v[pl.ds(i * 16, 16)]
            lane = lax.iota(jnp.int32, 16) + i * 16
            eidx_v[pl.ds(i * 16, 16)] = base + lane * _T + t16
            tidx_v[pl.ds(i * 16, 16)] = t16 * _T + tn16

        g1 = pltpu.async_copy(emis_hbm.at[eidx_v], evals_v, sem)
        g2 = pltpu.async_copy(trans_hbm.at[tidx_v], tvals_v, sem)

        @pl.when(wid == 0)
        def _():
            i16_v[...] = lax.iota(jnp.int32, 16) * _L
            pltpu.sync_copy(tags_hbm.at[i16_v], t16_v)
            pltpu.sync_copy(start_hbm.at[t16_v], f16_v)
            pltpu.sync_copy(f16_v, out_hbm.at[32])

        @pl.when(wid == 2)
        def _():
            pltpu.sync_copy(len_hbm, l16_v)
            i16_v[...] = lax.iota(jnp.int32, 16) * _L + l16_v[...] - 1
            pltpu.sync_copy(tags_hbm.at[i16_v], t16_v)
            pltpu.sync_copy(end_hbm.at[t16_v], f16_v)
            pltpu.sync_copy(f16_v, out_hbm.at[33])

        g1.wait()
        g2.wait()

        acc = jnp.zeros((16,), jnp.float32)
        for i in range(_HALF // 16):
            t_abs = lax.iota(jnp.int32, 16) + i * 16 + lo
            ev = evals_v[pl.ds(i * 16, 16)]
            tv = tvals_v[pl.ds(i * 16, 16)]
            acc = acc + jnp.where(t_abs < len_vec, ev, 0.0)
            m_tr = ((t_abs + 1) < len_vec) & (t_abs < (_L - 1))
            acc = acc + jnp.where(m_tr, tv, 0.0)
        acc_v[...] = acc
        pltpu.sync_copy(acc_v, out_hbm.at[wid])

    return pl.kernel(
        body,
        out_type=jax.ShapeDtypeStruct((34, 16), jnp.float32),
        mesh=mesh,
        scratch_types=[
            pltpu.VMEM((_HALF,), jnp.int32),
            pltpu.VMEM((_HALF,), jnp.int32),
            pltpu.VMEM((16,), jnp.int32),
            pltpu.VMEM((_HALF,), jnp.int32),
            pltpu.VMEM((_HALF,), jnp.int32),
            pltpu.VMEM((_HALF,), jnp.float32),
            pltpu.VMEM((_HALF,), jnp.float32),
            pltpu.VMEM((16,), jnp.float32),
            pltpu.VMEM((16,), jnp.int32),
            pltpu.VMEM((16,), jnp.int32),
            pltpu.VMEM((16,), jnp.float32),
            pltpu.VMEM((16,), jnp.int32),
            pltpu.SemaphoreType.DMA,
        ],
    )


def kernel(emissions, tags, lengths, transitions, start_transitions,
           end_transitions):
    lens = jnp.maximum(lengths, 1).astype(jnp.int32)

    tags_flat = tags.reshape(-1)
    tagsn_flat = jnp.concatenate(
        [tags[:, 1:], jnp.zeros((_B, 1), jnp.int32)], axis=1).reshape(-1)
    partials = _make_score_kernel()(
        emissions.reshape(-1), tags_flat, tagsn_flat, jnp.repeat(lens, 16),
        lens, transitions.reshape(-1), start_transitions, end_transitions)
    log_s = (jnp.sum(partials[:32].reshape(_B, 2 * 16), axis=1)
             + partials[32] + partials[33])

    emis_t = jnp.transpose(emissions, (1, 0, 2))
    log_z = pl.pallas_call(
        _partition_body,
        out_shape=jax.ShapeDtypeStruct((_B, 1), jnp.float32),
        scratch_shapes=[pltpu.VMEM((_L, _B, _W), jnp.bfloat16)],
    )(emis_t, lens[:, None], transitions,
      jnp.transpose(transitions, (1, 0)),
      start_transitions[None, :], end_transitions[None, :])

    return log_s - log_z[:, 0]
